# baseline (speedup 1.0000x reference)
import dataclasses

import jax
import jax.numpy as jnp
from jax import lax
from jax.experimental import pallas as pl
from jax.experimental.pallas import tpu as pltpu
from jax.experimental.pallas import tpu_sc as plsc

N = 320000
D = 128
L = 16
NW = 32
NSUB = 16
ROWS_PER_W = N // NW
BLK = 400
NBLK = ROWS_PER_W // BLK
NGRP = BLK // L
ZCH = 128
NCH = (BLK + ZCH - 1) // ZCH
SHROWS = NSUB * 2 * BLK


def _compiler_params():
    cp = pltpu.CompilerParams()
    if "needs_layout_passes" in pltpu.CompilerParams.__dataclass_fields__:
        cp = dataclasses.replace(cp, needs_layout_passes=False)
    return cp


def _sc_mask_rows(score_flat, y):
    mesh = plsc.VectorSubcoreMesh(core_axis_name="core", subcore_axis_name="subcore")

    @pl.kernel(
        out_type=jax.ShapeDtypeStruct((N, D), jnp.float32),
        mesh=mesh,
        scratch_types=[
            pltpu.VMEM_SHARED((SHROWS, D), jnp.float32),
            pltpu.VMEM((2 * NCH, ZCH), jnp.int32),
            pltpu.VMEM((ZCH, D), jnp.float32),
            pltpu.VMEM((BLK * 4,), jnp.float32),
            pltpu.VMEM((BLK * 4,), jnp.float32),
            pltpu.SemaphoreType.DMA,
            pltpu.SemaphoreType.DMA,
            pltpu.SemaphoreType.DMA,
            pltpu.SemaphoreType.DMA,
            pltpu.SemaphoreType.DMA,
            pltpu.SemaphoreType.DMA,
            pltpu.SemaphoreType.DMA,
        ],
        compiler_params=_compiler_params(),
    )
    def sc_kernel(
        score_hbm, y_hbm, o_hbm,
        shbuf, didx, zbuf, sb0, sb1,
        yi0, yi1, so0, so1, zsem, si0, si1,
    ):
        sub = lax.axis_index("subcore")
        wid = sub * 2 + lax.axis_index("core")
        base = wid * ROWS_PER_W
        sfbase = base * 4
        slot0 = sub * (2 * BLK)
        iota = lax.iota(jnp.int32, L)
        iota4 = iota * 4
        zero_v = jnp.zeros((L,), jnp.float32)

        @pl.loop(0, ZCH)
        def _(r):
            for c in range(D // L):
                zbuf[r, pl.ds(c * L, L)] = zero_v

        yins = (yi0, yi1)
        souts = (so0, so1)
        sbufs = (sb0, sb1)
        sins = (si0, si1)

        def start_in(i, b):
            cy = pltpu.make_async_copy(
                y_hbm.at[pl.ds(base + i * BLK, BLK), :],
                shbuf.at[pl.ds(slot0 + b * BLK, BLK), :],
                yins[b],
            )
            cy.start()
            cs = pltpu.make_async_copy(
                score_hbm.at[pl.ds(sfbase + i * (BLK * 4), BLK * 4)],
                sbufs[b],
                sins[b],
            )
            cs.start()
            return cy, cs

        def mk_out(i, b):
            return pltpu.make_async_copy(
                shbuf.at[pl.ds(slot0 + b * BLK, BLK), :],
                o_hbm.at[pl.ds(base + i * BLK, BLK), :],
                souts[b],
            )

        in_copies = [None, None]
        out_copies = [None, None]
        in_copies[0] = start_in(0, 0)

        for i in range(NBLK):
            b = i % 2
            nb = (i + 1) % 2
            if out_copies[nb] is not None:
                out_copies[nb].wait()
                out_copies[nb] = None
            if i + 1 < NBLK:
                in_copies[nb] = start_in(i + 1, nb)
            cy, cs = in_copies[b]
            cy.wait()
            cs.wait()

            sbuf = sbufs[b]
            srow0 = slot0 + b * BLK
            ch0 = b * NCH
            ch0_v = jnp.full((L,), ch0, jnp.int32)

            def grp_body(g, nd, sbuf=sbuf, srow0=srow0, ch0_v=ch0_v):
                gbase = jnp.full((L,), g * (L * 4), jnp.int32) + iota4
                c0 = plsc.load_gather(sbuf, [gbase])
                c1 = plsc.load_gather(sbuf, [gbase + 1])
                c2 = plsc.load_gather(sbuf, [gbase + 2])
                c3 = plsc.load_gather(sbuf, [gbase + 3])
                ss = c0 * c0 + c1 * c1 + c2 * c2 + c3 * c3
                drop = ss < 4.0
                rowg = jnp.full((L,), srow0, jnp.int32) + g * L + iota
                dpos = nd + plsc.cumsum(drop.astype(jnp.int32)) - 1
                plsc.store_scatter(
                    didx, [ch0_v + (dpos >> 7), dpos & 127], rowg, mask=drop
                )
                return nd + plsc.all_reduce_population_count(drop)

            nd = lax.fori_loop(0, NGRP, grp_body, jnp.zeros((L,), jnp.int32))

            nd_s = jnp.max(nd)
            nch = lax.shift_right_logical(nd_s + (ZCH - 1), 7)
            dend = lax.shift_left(nch, 7)
            d0 = plsc.load_gather(didx, [ch0_v, jnp.zeros((L,), jnp.int32)])
            for t in range(ZCH // L):
                dposs = nd + iota + t * L
                plsc.store_scatter(
                    didx,
                    [ch0_v + (dposs >> 7), dposs & 127],
                    d0,
                    mask=dposs < dend,
                )

            def zfire(j, c):
                pltpu.make_async_copy(zbuf, shbuf.at[didx.at[ch0 + j]], zsem).start()
                return c

            lax.fori_loop(0, nch, zfire, 0)

            def zdrain(j, c):
                pltpu.make_async_copy(zbuf, shbuf.at[didx.at[ch0]], zsem).wait()
                return c

            lax.fori_loop(0, nch, zdrain, 0)

            cout = mk_out(i, b)
            cout.start()
            out_copies[b] = cout

        for b in range(2):
            if out_copies[b] is not None:
                out_copies[b].wait()

    return sc_kernel(score_flat, y)


def kernel(edge_index, score, y):
    del edge_index
    score_flat = score.reshape(N * 4)
    return _sc_mask_rows(score_flat, y)
